# baseline (speedup 1.0000x reference)
import functools

import jax
import jax.numpy as jnp
import numpy as np
from jax import lax
from jax.experimental import pallas as pl
from jax.experimental.pallas import tpu as pltpu
from jax.experimental.pallas import tpu_sc as plsc

N = 2049
B = 8
TR = 512
NT = (N + TR - 1) // TR
PAD = 2064
NO = 2048

MATCH_THRESHOLD_F32 = np.float32(0.2)
BIG_I32 = np.int32(2**30)

SC_CORES = 2
SC_SUBCORES = 16
SC_LANES = 16
NW = SC_CORES * SC_SUBCORES
WPR = NW // B
CPW = NO // WPR


def _stage1_body(x_ref, rowmax_ref, rowarg_ref, colmax_ref, colarg_ref):
    t = pl.program_id(2)
    x = x_ref[0, 0]

    col_ids = lax.broadcasted_iota(jnp.int32, (TR, N), 1)
    rmax = jnp.max(x, axis=1, keepdims=True)
    rarg = jnp.min(jnp.where(x == rmax, col_ids, BIG_I32), axis=1, keepdims=True)
    rowmax_ref[0, 0] = rmax.T
    rowarg_ref[0, 0] = rarg.T

    @pl.when(t == 0)
    def _():
        row_ids = lax.broadcasted_iota(jnp.int32, (TR, N), 0)
        cmax = jnp.max(x, axis=0, keepdims=True)
        carg = jnp.min(jnp.where(x == cmax, row_ids, BIG_I32), axis=0, keepdims=True)
        colmax_ref[0, 0] = cmax
        colarg_ref[0, 0] = carg

    @pl.when(jnp.logical_and(t > 0, t < NT - 1))
    def _():
        row_ids = lax.broadcasted_iota(jnp.int32, (TR, N), 0) + t * TR
        cmax = jnp.max(x, axis=0, keepdims=True)
        carg = jnp.min(jnp.where(x == cmax, row_ids, BIG_I32), axis=0, keepdims=True)
        prev_max = colmax_ref[0, 0]
        prev_arg = colarg_ref[0, 0]
        upd = cmax > prev_max
        colmax_ref[0, 0] = jnp.where(upd, cmax, prev_max)
        colarg_ref[0, 0] = jnp.where(upd, carg, prev_arg)

    @pl.when(t == NT - 1)
    def _():
        last = x[0:1, :]
        prev_max = colmax_ref[0, 0]
        prev_arg = colarg_ref[0, 0]
        upd = last > prev_max
        colmax_ref[0, 0] = jnp.where(upd, last, prev_max)
        colarg_ref[0, 0] = jnp.where(upd, jnp.full_like(prev_arg, N - 1), prev_arg)


_stage1 = pl.pallas_call(
    _stage1_body,
    grid=(2, 4, NT),
    in_specs=[pl.BlockSpec((1, 1, TR, N), lambda a, b, t: (a, b, t, 0))],
    out_specs=[
        pl.BlockSpec((1, 1, 1, TR), lambda a, b, t: (a, b, 0, t)),
        pl.BlockSpec((1, 1, 1, TR), lambda a, b, t: (a, b, 0, t)),
        pl.BlockSpec((1, 1, 1, N), lambda a, b, t: (a, b, 0, 0)),
        pl.BlockSpec((1, 1, 1, N), lambda a, b, t: (a, b, 0, 0)),
    ],
    out_shape=[
        jax.ShapeDtypeStruct((2, 4, 1, N), jnp.float32),
        jax.ShapeDtypeStruct((2, 4, 1, N), jnp.int32),
        jax.ShapeDtypeStruct((2, 4, 1, N), jnp.float32),
        jax.ShapeDtypeStruct((2, 4, 1, N), jnp.int32),
    ],
    compiler_params=pltpu.CompilerParams(
        dimension_semantics=("parallel", "parallel", "arbitrary"),
    ),
)


@functools.partial(
    pl.kernel,
    out_type=jax.ShapeDtypeStruct((B, NO), jnp.int32),
    mesh=plsc.VectorSubcoreMesh(core_axis_name="c", subcore_axis_name="s"),
    compiler_params=pltpu.CompilerParams(needs_layout_passes=False),
    scratch_types=[
        pltpu.VMEM((PAD,), jnp.int32),
        pltpu.VMEM((CPW,), jnp.int32),
        pltpu.VMEM((CPW,), jnp.float32),
        pltpu.VMEM((CPW,), jnp.int32),
    ],
)
def _stage2(max0_hbm, m0_hbm, m1_hbm, out_hbm, m1row_v, m0_v, mx_v, out_v):
    wid = lax.axis_index("s") * SC_CORES + lax.axis_index("c")
    p = wid // WPR
    base = (wid % WPR) * CPW
    pltpu.sync_copy(m1_hbm.at[p], m1row_v)
    pltpu.sync_copy(m0_hbm.at[p, pl.ds(base, CPW)], m0_v)
    pltpu.sync_copy(max0_hbm.at[p, pl.ds(base, CPW)], mx_v)
    for k in range(CPW // SC_LANES):
        off = k * SC_LANES
        idx = m0_v[pl.ds(off, SC_LANES)]
        g = plsc.load_gather(m1row_v, [idx])
        r = base + off + lax.iota(jnp.int32, SC_LANES)
        mutual = g == r
        ok = jnp.logical_and(mutual, mx_v[pl.ds(off, SC_LANES)] > MATCH_THRESHOLD_F32)
        out_v[pl.ds(off, SC_LANES)] = jnp.where(ok, idx, np.int32(-1))
    pltpu.sync_copy(out_v, out_hbm.at[p, pl.ds(base, CPW)])


@jax.jit
def kernel(scores_list):
    rowmax, rowarg, _, colarg = _stage1(scores_list)
    pad = ((0, 0), (0, PAD - N))
    max0 = jnp.pad(rowmax.reshape(B, N), pad)
    m0 = jnp.pad(rowarg.reshape(B, N), pad)
    m1 = jnp.pad(colarg.reshape(B, N), pad)
    out = _stage2(max0, m0, m1).reshape(2, 4, NO)
    return (out[0], out[1])
